# Initial kernel scaffold; baseline (speedup 1.0000x reference)
import jax
import jax.numpy as jnp
from jax.experimental import pallas as pl


def kernel(node_features, edge_attr, weather_attr, edge_index, node_batch, edge_batch, Wn1, asn1, adn1, bn1, Wn2, asn2, adn2, bn2, We1, ase1, ade1, be1, We2, ase2, ade2, be2, Wm1, bm1, Wm2, bm2, g_nbn1, b_nbn1, g_nbn2, b_nbn2, g_ebn1, b_ebn1, g_ebn2, b_ebn2, g_wbn, b_wbn, Wr1, br1, Wr2, br2):
    raise NotImplementedError("write your pallas kernel here")


# speedup vs baseline: 31.5262x; 31.5262x over previous
import functools

import jax
import jax.numpy as jnp
from jax import lax
from jax.experimental import pallas as pl
from jax.experimental.pallas import tpu as pltpu
from jax.experimental.pallas import tpu_sc as plsc

N = 10000
E = 320000
GRP = 16
H = 64

NC, NS, L = 2, 16, 16
NW = NC * NS
K = 128
N_IT = 81
T_PER = K * N_IT
T_ALL = NW * T_PER
NP = 10240
STRIPE = NP // NS
TRASH = N

_f32 = jnp.float32
_HIGH = lax.Precision.HIGHEST


def _edge_body(src_hbm, dst_hbm, hn_hbm, he_hbm,
               alsn_hbm, aldn_hbm, alse_hbm, alde_hbm,
               usum_n_out, usum_e_out, den_n_out, den_e_out,
               src_t, dst_t, als_t, ald_t,
               rows, wbuf, zrows, zw,
               usum_sp, den_sp, sem1):
    c = lax.axis_index("c")
    s = lax.axis_index("s")
    wid = c * NS + s
    base = s * STRIPE

    pltpu.sync_copy(src_hbm.at[wid], src_t)
    pltpu.sync_copy(dst_hbm.at[wid], dst_t)

    zero16 = jnp.zeros((L,), _f32)

    def zrow(r, _):
        for j in range(H // L):
            zrows[r, pl.ds(j * L, L)] = zero16
        return 0
    lax.fori_loop(jnp.int32(0), jnp.int32(K), zrow, 0)
    for g in range(K // L):
        zw[pl.ds(g * L, L)] = zero16

    for h_hbm, als_hbm, ald_hbm, usum_out, den_out in (
            (hn_hbm, alsn_hbm, aldn_hbm, usum_n_out, den_n_out),
            (he_hbm, alse_hbm, alde_hbm, usum_e_out, den_e_out)):
        pltpu.sync_copy(als_hbm, als_t)
        pltpu.sync_copy(ald_hbm, ald_t)
        for b in range(STRIPE // K):
            pltpu.sync_copy(zrows, usum_sp.at[pl.ds(base + b * K, K)])
            pltpu.sync_copy(zw, den_sp.at[pl.ds(base + b * K, K)])
        plsc.subcore_barrier()

        def chunk(it, _):
            pltpu.async_copy(h_hbm.at[src_t.at[it]], rows, sem1).wait()
            for g in range(K // L):
                sl = pl.ds(g * L, L)
                s16 = src_t[it, sl]
                d16 = dst_t[it, sl]
                e = plsc.load_gather(als_t, [s16]) + \
                    plsc.load_gather(ald_t, [d16])
                wbuf[sl] = jnp.exp(jnp.maximum(e, 0.2 * e))

            def scale(r, _):
                w = plsc.load_gather(wbuf, [jnp.full((L,), r)])
                for j in range(H // L):
                    sj = pl.ds(j * L, L)
                    rows[r, sj] = rows[r, sj] * w
                return 0
            lax.fori_loop(jnp.int32(0), jnp.int32(K), scale, 0)

            pltpu.sync_copy(rows, usum_sp.at[dst_t.at[it]], add=True)
            pltpu.sync_copy(wbuf, den_sp.at[dst_t.at[it]], add=True)
            return 0

        lax.fori_loop(jnp.int32(0), jnp.int32(N_IT), chunk, 0)
        plsc.subcore_barrier()

        for b in range(STRIPE // K):
            off = base + b * K
            pltpu.sync_copy(usum_sp.at[pl.ds(off, K)],
                            usum_out.at[c, pl.ds(off, K)])
            pltpu.sync_copy(den_sp.at[pl.ds(off, K)],
                            den_out.at[pl.ds(c * NP + off, K)])
        plsc.subcore_barrier()


@functools.cache
def _edge_kernel_fn():
  return pl.kernel(
    _edge_body,
    out_type=[jax.ShapeDtypeStruct((NC, NP, H), _f32),
              jax.ShapeDtypeStruct((NC, NP, H), _f32),
              jax.ShapeDtypeStruct((NC * NP,), _f32),
              jax.ShapeDtypeStruct((NC * NP,), _f32)],
    mesh=plsc.VectorSubcoreMesh(core_axis_name="c", subcore_axis_name="s"),
    compiler_params=pltpu.CompilerParams(needs_layout_passes=False,
                                         use_tc_tiling_on_sc=False),
    scratch_types=[
        pltpu.VMEM((N_IT, K), jnp.int32),
        pltpu.VMEM((N_IT, K), jnp.int32),
        pltpu.VMEM((NP,), _f32),
        pltpu.VMEM((NP,), _f32),
        pltpu.VMEM((K, H), _f32),
        pltpu.VMEM((K,), _f32),
        pltpu.VMEM((K, H), _f32),
        pltpu.VMEM((K,), _f32),
        pltpu.VMEM_SHARED((NP, H), _f32),
        pltpu.VMEM_SHARED((NP,), _f32),
        pltpu.SemaphoreType.DMA,
    ],
  )


def _dense_a_body(nf_ref, ea_ref, Wn_ref, avn_ref, We_ref, ave_ref,
                  hn_ref, sn_ref, he_ref, se_ref):
    hn = jnp.dot(nf_ref[...], Wn_ref[...],
                 preferred_element_type=_f32, precision=_HIGH)
    hn_ref[...] = hn
    sn_ref[...] = jnp.dot(hn, avn_ref[...],
                          preferred_element_type=_f32, precision=_HIGH)
    he = jnp.dot(ea_ref[...], We_ref[...],
                 preferred_element_type=_f32, precision=_HIGH)
    he_ref[...] = he
    se_ref[...] = jnp.dot(he, ave_ref[...],
                          preferred_element_type=_f32, precision=_HIGH)


_dense_a = pl.pallas_call(
    _dense_a_body,
    out_shape=[jax.ShapeDtypeStruct((NP, H), _f32),
               jax.ShapeDtypeStruct((NP, 2), _f32),
               jax.ShapeDtypeStruct((NP, H), _f32),
               jax.ShapeDtypeStruct((NP, 2), _f32)],
)


def _combine_bn(usum, den2, bias, gg, bb, relu):
    dd = jnp.transpose(jnp.maximum(den2[0:1, :] + den2[1:2, :], 1e-30))
    o = (usum[0] + usum[1]) / dd + bias
    if relu:
        o = jnp.maximum(o, 0.0)
    rows = lax.broadcasted_iota(jnp.int32, (NP, 1), 0)
    mask = rows < N
    o = jnp.where(mask, o, 0.0)
    mean = jnp.sum(o, axis=0, keepdims=True) / N
    var = jnp.sum(o * o, axis=0, keepdims=True) / N - mean * mean
    xn = (o - mean) / jnp.sqrt(var + 1e-5) * gg + bb
    return jnp.where(mask, xn, 0.0)


def _dense_b_body(usum_ref, den_ref, bias_ref, g_ref, bb_ref, W2_ref,
                  av2_ref, h2_ref, s2_ref):
    xn = _combine_bn(usum_ref[...], den_ref[...], bias_ref[...],
                     g_ref[...], bb_ref[...], relu=True)
    h2 = jnp.dot(xn, W2_ref[...], preferred_element_type=_f32,
                 precision=_HIGH)
    h2_ref[...] = h2
    s2_ref[...] = jnp.dot(h2, av2_ref[...], preferred_element_type=_f32,
                          precision=_HIGH)


_dense_b = pl.pallas_call(
    _dense_b_body,
    out_shape=[jax.ShapeDtypeStruct((NP, H), _f32),
               jax.ShapeDtypeStruct((NP, 2), _f32)],
)


def _pool(x, batch_ref):
    b = batch_ref[...]
    gids = lax.broadcasted_iota(jnp.int32, (1, GRP), 1)
    onehot = (b == gids).astype(_f32)
    sums = lax.dot_general(onehot, x, (((0,), (0,)), ((), ())),
                           preferred_element_type=_f32,
                           precision=_HIGH)
    cnt = jnp.transpose(jnp.sum(onehot, axis=0, keepdims=True))
    return sums / jnp.maximum(cnt, 1.0)


def _dense_cpart_body(usum_ref, den_ref, bias_ref, g_ref, bb_ref, batch_ref,
                      pool_ref):
    xh = _combine_bn(usum_ref[...], den_ref[...], bias_ref[...],
                     g_ref[...], bb_ref[...], relu=False)
    pool_ref[...] = _pool(xh, batch_ref)


_dense_cpart = pl.pallas_call(
    _dense_cpart_body,
    out_shape=jax.ShapeDtypeStruct((GRP, H), _f32),
)


def _dense_cfin_body(pn_ref, pe_ref, w_ref, Wm1_ref, bm1_ref, gw_ref, bw_ref,
                     Wm2_ref, bm2_ref, Wr1_ref, br1_ref, Wr2_ref, br2_ref,
                     out_ref):
    wh = jnp.dot(w_ref[...], Wm1_ref[...], preferred_element_type=_f32,
                 precision=_HIGH) + bm1_ref[...]
    wh = jnp.maximum(wh, 0.0)
    m = jnp.mean(wh, axis=0, keepdims=True)
    v = jnp.mean(wh * wh, axis=0, keepdims=True) - m * m
    wh = (wh - m) / jnp.sqrt(v + 1e-5) * gw_ref[...] + bw_ref[...]
    wh = jnp.dot(wh, Wm2_ref[...], preferred_element_type=_f32,
                 precision=_HIGH) + bm2_ref[...]
    comb = jnp.concatenate([pn_ref[...], pe_ref[...], wh], axis=-1)
    oh = jnp.dot(comb, Wr1_ref[...], preferred_element_type=_f32,
                 precision=_HIGH) + br1_ref[...]
    oh = jnp.maximum(oh, 0.0)
    out = jnp.dot(oh, Wr2_ref[...], preferred_element_type=_f32,
                  precision=_HIGH) + br2_ref[...]
    out_ref[...] = jax.nn.sigmoid(out)


_dense_cfin = pl.pallas_call(
    _dense_cfin_body,
    out_shape=jax.ShapeDtypeStruct((GRP, 1), _f32),
)


def kernel(node_features, edge_attr, weather_attr, edge_index, node_batch,
           edge_batch, Wn1, asn1, adn1, bn1, Wn2, asn2, adn2, bn2,
           We1, ase1, ade1, be1, We2, ase2, ade2, be2,
           Wm1, bm1, Wm2, bm2,
           g_nbn1, b_nbn1, g_nbn2, b_nbn2, g_ebn1, b_ebn1, g_ebn2, b_ebn2,
           g_wbn, b_wbn, Wr1, br1, Wr2, br2):
    f32 = _f32
    loops = jnp.arange(N, dtype=jnp.int32)
    src = jnp.concatenate([edge_index[0].astype(jnp.int32), loops])
    dst = jnp.concatenate([edge_index[1].astype(jnp.int32), loops])
    pad = T_ALL - src.shape[0]
    src = jnp.concatenate([src, jnp.zeros((pad,), jnp.int32)])
    dst = jnp.concatenate([dst, jnp.full((pad,), TRASH, jnp.int32)])
    src3 = src.reshape(NW, N_IT, K)
    dst3 = dst.reshape(NW, N_IT, K)

    nf_p = jnp.zeros((NP, node_features.shape[1]), f32).at[:N].set(
        node_features.astype(f32))
    ea_p = jnp.zeros((NP, edge_attr.shape[1]), f32).at[:N].set(
        edge_attr.astype(f32))
    nb_p = jnp.full((NP, 1), GRP, jnp.int32).at[:N, 0].set(
        node_batch.astype(jnp.int32))
    eb_p = jnp.full((NP, 1), GRP, jnp.int32).at[:N, 0].set(
        edge_batch.astype(jnp.int32))

    def row(v):
        return v.astype(f32).reshape(1, -1)

    def av(a, d):
        return jnp.stack([a.astype(f32), d.astype(f32)], axis=1)

    hn1, sn1, he1, se1 = _dense_a(nf_p, ea_p, Wn1.astype(f32), av(asn1, adn1),
                                  We1.astype(f32), av(ase1, ade1))
    un1, ue1, dn1, de1 = _edge_kernel_fn()(src3, dst3, hn1, he1,
                                      sn1[:, 0], sn1[:, 1],
                                      se1[:, 0], se1[:, 1])
    hn2, sn2 = _dense_b(un1, dn1.reshape(NC, NP),
                        row(bn1), row(g_nbn1), row(b_nbn1),
                        Wn2.astype(f32), av(asn2, adn2))
    he2, se2 = _dense_b(ue1, de1.reshape(NC, NP),
                        row(be1), row(g_ebn1), row(b_ebn1),
                        We2.astype(f32), av(ase2, ade2))
    un2, ue2, dn2, de2 = _edge_kernel_fn()(src3, dst3, hn2, he2,
                                      sn2[:, 0], sn2[:, 1],
                                      se2[:, 0], se2[:, 1])
    pn = _dense_cpart(un2, dn2.reshape(NC, NP),
                      row(bn2), row(g_nbn2), row(b_nbn2), nb_p)
    pe = _dense_cpart(ue2, de2.reshape(NC, NP),
                      row(be2), row(g_ebn2), row(b_ebn2), eb_p)
    out = _dense_cfin(
        pn, pe, weather_attr.astype(f32), Wm1.astype(f32), row(bm1),
        row(g_wbn), row(b_wbn), Wm2.astype(f32), row(bm2),
        Wr1.astype(f32), row(br1), Wr2.astype(f32), row(br2))
    return out


# speedup vs baseline: 35.4022x; 1.1229x over previous
import functools

import jax
import jax.numpy as jnp
from jax import lax
from jax.experimental import pallas as pl
from jax.experimental.pallas import tpu as pltpu
from jax.experimental.pallas import tpu_sc as plsc

N = 10000
E = 320000
GRP = 16
H = 64

NC, NS, L = 2, 16, 16
NW = NC * NS
K = 128
N_IT = 82
T_PER = K * N_IT
T_ALL = NW * T_PER
NP = 10240
STRIPE = NP // NS
TRASH = N

_f32 = jnp.float32
_HIGH = lax.Precision.HIGHEST


def _edge_body(src_hbm, dst_hbm, hn_hbm, he_hbm,
               alsn_hbm, aldn_hbm, alse_hbm, alde_hbm,
               usum_n_out, usum_e_out, den_n_out, den_e_out,
               src_t, dst_t, als_t, ald_t,
               rows_a, rows_b, wbuf, zrows, zw,
               usum_sp, den_sp, sem_a, sem_b):
    c = lax.axis_index("c")
    s = lax.axis_index("s")
    wid = c * NS + s
    base = s * STRIPE

    pltpu.sync_copy(src_hbm.at[wid], src_t)
    pltpu.sync_copy(dst_hbm.at[wid], dst_t)

    zero16 = jnp.zeros((L,), _f32)

    def zrow(r, _):
        for j in range(H // L):
            zrows[r, pl.ds(j * L, L)] = zero16
        return 0
    lax.fori_loop(jnp.int32(0), jnp.int32(K), zrow, 0)
    for g in range(K // L):
        zw[pl.ds(g * L, L)] = zero16

    for h_hbm, als_hbm, ald_hbm, usum_out, den_out in (
            (hn_hbm, alsn_hbm, aldn_hbm, usum_n_out, den_n_out),
            (he_hbm, alse_hbm, alde_hbm, usum_e_out, den_e_out)):
        pltpu.sync_copy(als_hbm, als_t)
        pltpu.sync_copy(ald_hbm, ald_t)
        for b in range(STRIPE // K):
            pltpu.sync_copy(zrows, usum_sp.at[pl.ds(base + b * K, K)])
            pltpu.sync_copy(zw, den_sp.at[pl.ds(base + b * K, K)])
        plsc.subcore_barrier()

        def process(it, rows):
            for g in range(K // L):
                sl = pl.ds(g * L, L)
                s16 = src_t[it, sl]
                d16 = dst_t[it, sl]
                e = plsc.load_gather(als_t, [s16]) + \
                    plsc.load_gather(ald_t, [d16])
                wbuf[sl] = jnp.exp(jnp.maximum(e, 0.2 * e))

            @plsc.parallel_loop(jnp.int32(0), jnp.int32(K), step=jnp.int32(1), unroll=2)
            def _scale(r):
                w = plsc.load_gather(wbuf, [jnp.full((L,), r)])
                for j in range(H // L):
                    sj = pl.ds(j * L, L)
                    rows[r, sj] = rows[r, sj] * w

            pltpu.sync_copy(rows, usum_sp.at[dst_t.at[it]], add=True)
            pltpu.sync_copy(wbuf, den_sp.at[dst_t.at[it]], add=True)

        pltpu.async_copy(h_hbm.at[src_t.at[jnp.int32(0)]], rows_a, sem_a)

        def pair(i2, _):
            it0 = i2 * 2
            it1 = it0 + 1
            pltpu.make_async_copy(h_hbm.at[src_t.at[it0]], rows_a,
                                  sem_a).wait()
            pltpu.async_copy(h_hbm.at[src_t.at[it1]], rows_b, sem_b)
            process(it0, rows_a)
            pltpu.make_async_copy(h_hbm.at[src_t.at[it1]], rows_b,
                                  sem_b).wait()

            @pl.when(it1 + 1 < N_IT)
            def _():
                pltpu.async_copy(h_hbm.at[src_t.at[it1 + 1]], rows_a, sem_a)
            process(it1, rows_b)
            return 0

        lax.fori_loop(jnp.int32(0), jnp.int32(N_IT // 2), pair, 0)
        plsc.subcore_barrier()

        for b in range(STRIPE // K):
            off = base + b * K
            pltpu.sync_copy(usum_sp.at[pl.ds(off, K)],
                            usum_out.at[c, pl.ds(off, K)])
            pltpu.sync_copy(den_sp.at[pl.ds(off, K)],
                            den_out.at[pl.ds(c * NP + off, K)])
        plsc.subcore_barrier()


@functools.cache
def _edge_kernel_fn():
  return pl.kernel(
    _edge_body,
    out_type=[jax.ShapeDtypeStruct((NC, NP, H), _f32),
              jax.ShapeDtypeStruct((NC, NP, H), _f32),
              jax.ShapeDtypeStruct((NC * NP,), _f32),
              jax.ShapeDtypeStruct((NC * NP,), _f32)],
    mesh=plsc.VectorSubcoreMesh(core_axis_name="c", subcore_axis_name="s"),
    compiler_params=pltpu.CompilerParams(needs_layout_passes=False,
                                         use_tc_tiling_on_sc=False),
    scratch_types=[
        pltpu.VMEM((N_IT, K), jnp.int32),
        pltpu.VMEM((N_IT, K), jnp.int32),
        pltpu.VMEM((NP,), _f32),
        pltpu.VMEM((NP,), _f32),
        pltpu.VMEM((K, H), _f32),
        pltpu.VMEM((K, H), _f32),
        pltpu.VMEM((K,), _f32),
        pltpu.VMEM((K, H), _f32),
        pltpu.VMEM((K,), _f32),
        pltpu.VMEM_SHARED((NP, H), _f32),
        pltpu.VMEM_SHARED((NP,), _f32),
        pltpu.SemaphoreType.DMA,
        pltpu.SemaphoreType.DMA,
    ],
  )


def _dense_a_body(nf_ref, ea_ref, Wn_ref, avn_ref, We_ref, ave_ref,
                  hn_ref, sn_ref, he_ref, se_ref):
    hn = jnp.dot(nf_ref[...], Wn_ref[...],
                 preferred_element_type=_f32, precision=_HIGH)
    hn_ref[...] = hn
    sn_ref[...] = jnp.dot(hn, avn_ref[...],
                          preferred_element_type=_f32, precision=_HIGH)
    he = jnp.dot(ea_ref[...], We_ref[...],
                 preferred_element_type=_f32, precision=_HIGH)
    he_ref[...] = he
    se_ref[...] = jnp.dot(he, ave_ref[...],
                          preferred_element_type=_f32, precision=_HIGH)


_dense_a = pl.pallas_call(
    _dense_a_body,
    out_shape=[jax.ShapeDtypeStruct((NP, H), _f32),
               jax.ShapeDtypeStruct((NP, 2), _f32),
               jax.ShapeDtypeStruct((NP, H), _f32),
               jax.ShapeDtypeStruct((NP, 2), _f32)],
)


def _combine_bn(usum, den2, bias, gg, bb, relu):
    dd = jnp.transpose(jnp.maximum(den2[0:1, :] + den2[1:2, :], 1e-30))
    o = (usum[0] + usum[1]) / dd + bias
    if relu:
        o = jnp.maximum(o, 0.0)
    rows = lax.broadcasted_iota(jnp.int32, (NP, 1), 0)
    mask = rows < N
    o = jnp.where(mask, o, 0.0)
    mean = jnp.sum(o, axis=0, keepdims=True) / N
    var = jnp.sum(o * o, axis=0, keepdims=True) / N - mean * mean
    xn = (o - mean) / jnp.sqrt(var + 1e-5) * gg + bb
    return jnp.where(mask, xn, 0.0)


def _dense_b_body(usum_ref, den_ref, bias_ref, g_ref, bb_ref, W2_ref,
                  av2_ref, h2_ref, s2_ref):
    xn = _combine_bn(usum_ref[...], den_ref[...], bias_ref[...],
                     g_ref[...], bb_ref[...], relu=True)
    h2 = jnp.dot(xn, W2_ref[...], preferred_element_type=_f32,
                 precision=_HIGH)
    h2_ref[...] = h2
    s2_ref[...] = jnp.dot(h2, av2_ref[...], preferred_element_type=_f32,
                          precision=_HIGH)


_dense_b = pl.pallas_call(
    _dense_b_body,
    out_shape=[jax.ShapeDtypeStruct((NP, H), _f32),
               jax.ShapeDtypeStruct((NP, 2), _f32)],
)


def _pool(x, batch_ref):
    b = batch_ref[...]
    gids = lax.broadcasted_iota(jnp.int32, (1, GRP), 1)
    onehot = (b == gids).astype(_f32)
    sums = lax.dot_general(onehot, x, (((0,), (0,)), ((), ())),
                           preferred_element_type=_f32,
                           precision=_HIGH)
    cnt = jnp.transpose(jnp.sum(onehot, axis=0, keepdims=True))
    return sums / jnp.maximum(cnt, 1.0)


def _dense_cpart_body(usum_ref, den_ref, bias_ref, g_ref, bb_ref, batch_ref,
                      pool_ref):
    xh = _combine_bn(usum_ref[...], den_ref[...], bias_ref[...],
                     g_ref[...], bb_ref[...], relu=False)
    pool_ref[...] = _pool(xh, batch_ref)


_dense_cpart = pl.pallas_call(
    _dense_cpart_body,
    out_shape=jax.ShapeDtypeStruct((GRP, H), _f32),
)


def _dense_cfin_body(pn_ref, pe_ref, w_ref, Wm1_ref, bm1_ref, gw_ref, bw_ref,
                     Wm2_ref, bm2_ref, Wr1_ref, br1_ref, Wr2_ref, br2_ref,
                     out_ref):
    wh = jnp.dot(w_ref[...], Wm1_ref[...], preferred_element_type=_f32,
                 precision=_HIGH) + bm1_ref[...]
    wh = jnp.maximum(wh, 0.0)
    m = jnp.mean(wh, axis=0, keepdims=True)
    v = jnp.mean(wh * wh, axis=0, keepdims=True) - m * m
    wh = (wh - m) / jnp.sqrt(v + 1e-5) * gw_ref[...] + bw_ref[...]
    wh = jnp.dot(wh, Wm2_ref[...], preferred_element_type=_f32,
                 precision=_HIGH) + bm2_ref[...]
    comb = jnp.concatenate([pn_ref[...], pe_ref[...], wh], axis=-1)
    oh = jnp.dot(comb, Wr1_ref[...], preferred_element_type=_f32,
                 precision=_HIGH) + br1_ref[...]
    oh = jnp.maximum(oh, 0.0)
    out = jnp.dot(oh, Wr2_ref[...], preferred_element_type=_f32,
                  precision=_HIGH) + br2_ref[...]
    out_ref[...] = jax.nn.sigmoid(out)


_dense_cfin = pl.pallas_call(
    _dense_cfin_body,
    out_shape=jax.ShapeDtypeStruct((GRP, 1), _f32),
)


def kernel(node_features, edge_attr, weather_attr, edge_index, node_batch,
           edge_batch, Wn1, asn1, adn1, bn1, Wn2, asn2, adn2, bn2,
           We1, ase1, ade1, be1, We2, ase2, ade2, be2,
           Wm1, bm1, Wm2, bm2,
           g_nbn1, b_nbn1, g_nbn2, b_nbn2, g_ebn1, b_ebn1, g_ebn2, b_ebn2,
           g_wbn, b_wbn, Wr1, br1, Wr2, br2):
    f32 = _f32
    loops = jnp.arange(N, dtype=jnp.int32)
    src = jnp.concatenate([edge_index[0].astype(jnp.int32), loops])
    dst = jnp.concatenate([edge_index[1].astype(jnp.int32), loops])
    pad = T_ALL - src.shape[0]
    src = jnp.concatenate([src, jnp.zeros((pad,), jnp.int32)])
    dst = jnp.concatenate([dst, jnp.full((pad,), TRASH, jnp.int32)])
    src3 = src.reshape(NW, N_IT, K)
    dst3 = dst.reshape(NW, N_IT, K)

    nf_p = jnp.zeros((NP, node_features.shape[1]), f32).at[:N].set(
        node_features.astype(f32))
    ea_p = jnp.zeros((NP, edge_attr.shape[1]), f32).at[:N].set(
        edge_attr.astype(f32))
    nb_p = jnp.full((NP, 1), GRP, jnp.int32).at[:N, 0].set(
        node_batch.astype(jnp.int32))
    eb_p = jnp.full((NP, 1), GRP, jnp.int32).at[:N, 0].set(
        edge_batch.astype(jnp.int32))

    def row(v):
        return v.astype(f32).reshape(1, -1)

    def av(a, d):
        return jnp.stack([a.astype(f32), d.astype(f32)], axis=1)

    hn1, sn1, he1, se1 = _dense_a(nf_p, ea_p, Wn1.astype(f32), av(asn1, adn1),
                                  We1.astype(f32), av(ase1, ade1))
    un1, ue1, dn1, de1 = _edge_kernel_fn()(src3, dst3, hn1, he1,
                                      sn1[:, 0], sn1[:, 1],
                                      se1[:, 0], se1[:, 1])
    hn2, sn2 = _dense_b(un1, dn1.reshape(NC, NP),
                        row(bn1), row(g_nbn1), row(b_nbn1),
                        Wn2.astype(f32), av(asn2, adn2))
    he2, se2 = _dense_b(ue1, de1.reshape(NC, NP),
                        row(be1), row(g_ebn1), row(b_ebn1),
                        We2.astype(f32), av(ase2, ade2))
    un2, ue2, dn2, de2 = _edge_kernel_fn()(src3, dst3, hn2, he2,
                                      sn2[:, 0], sn2[:, 1],
                                      se2[:, 0], se2[:, 1])
    pn = _dense_cpart(un2, dn2.reshape(NC, NP),
                      row(bn2), row(g_nbn2), row(b_nbn2), nb_p)
    pe = _dense_cpart(ue2, de2.reshape(NC, NP),
                      row(be2), row(g_ebn2), row(b_ebn2), eb_p)
    out = _dense_cfin(
        pn, pe, weather_attr.astype(f32), Wm1.astype(f32), row(bm1),
        row(g_wbn), row(b_wbn), Wm2.astype(f32), row(bm2),
        Wr1.astype(f32), row(br1), Wr2.astype(f32), row(br2))
    return out


# speedup vs baseline: 55.0957x; 1.5563x over previous
import functools

import jax
import jax.numpy as jnp
from jax import lax
from jax.experimental import pallas as pl
from jax.experimental.pallas import tpu as pltpu
from jax.experimental.pallas import tpu_sc as plsc

N = 10000
E = 320000
GRP = 16
H = 64

NC, NS, L = 2, 16, 16
NW = NC * NS
K = 128
N_IT = 81
T_PER = K * N_IT
T_ALL = NW * T_PER
NP = 10240
STRIPE = NP // NS
TRASH = N

_f32 = jnp.float32
_HIGH = lax.Precision.HIGHEST


def _edge_body(src_hbm, dst_hbm, hn_hbm, he_hbm,
               alsn_hbm, aldn_hbm, alse_hbm, alde_hbm,
               usum_n_out, usum_e_out, den_n_out, den_e_out,
               src_t, dst_t, als_t, ald_t,
               rows0, rows1, rows2, wbuf0, wbuf1, wbuf2, zrows, zw,
               usum_sp, den_sp,
               sg0, sg1, sg2, su0, su1, su2, sd0, sd1, sd2):
    c = lax.axis_index("c")
    s = lax.axis_index("s")
    wid = c * NS + s
    base = s * STRIPE
    rows = (rows0, rows1, rows2)
    wbufs = (wbuf0, wbuf1, wbuf2)
    sg = (sg0, sg1, sg2)
    su = (su0, su1, su2)
    sd = (sd0, sd1, sd2)

    pltpu.sync_copy(src_hbm.at[wid], src_t)
    pltpu.sync_copy(dst_hbm.at[wid], dst_t)

    zero16 = jnp.zeros((L,), _f32)

    def zrow(r, _):
        for j in range(H // L):
            zrows[r, pl.ds(j * L, L)] = zero16
        return 0
    lax.fori_loop(jnp.int32(0), jnp.int32(K), zrow, 0)
    for g in range(K // L):
        zw[pl.ds(g * L, L)] = zero16

    for h_hbm, als_hbm, ald_hbm, usum_out, den_out in (
            (hn_hbm, alsn_hbm, aldn_hbm, usum_n_out, den_n_out),
            (he_hbm, alse_hbm, alde_hbm, usum_e_out, den_e_out)):
        pltpu.sync_copy(als_hbm, als_t)
        pltpu.sync_copy(ald_hbm, ald_t)
        for b in range(STRIPE // K):
            pltpu.sync_copy(zrows, usum_sp.at[pl.ds(base + b * K, K)])
            pltpu.sync_copy(zw, den_sp.at[pl.ds(base + b * K, K)])
        plsc.subcore_barrier()

        def gather(it, x):
            pltpu.async_copy(h_hbm.at[src_t.at[jnp.int32(it)]], rows[x],
                             sg[x])

        def compute(it, rbuf, wb):
            for g in range(K // L):
                sl = pl.ds(g * L, L)
                s16 = src_t[it, sl]
                d16 = dst_t[it, sl]
                e = plsc.load_gather(als_t, [s16]) + \
                    plsc.load_gather(ald_t, [d16])
                wb[sl] = jnp.exp(jnp.maximum(e, 0.2 * e))

            @plsc.parallel_loop(jnp.int32(0), jnp.int32(K),
                                step=jnp.int32(1), unroll=2)
            def _scale(r):
                w = plsc.load_gather(wb, [jnp.full((L,), r)])
                for j in range(H // L):
                    sj = pl.ds(j * L, L)
                    rbuf[r, sj] = rbuf[r, sj] * w

        def step(it, x, i3, first_of_phase):
            y = (x + 2) % 3
            pltpu.make_async_copy(h_hbm.at[src_t.at[jnp.int32(it)]],
                                  rows[x], sg[x]).wait()
            compute(it, rows[x], wbufs[x])
            pltpu.async_copy(rows[x], usum_sp.at[dst_t.at[jnp.int32(it)]],
                             su[x], add=True)
            pltpu.async_copy(wbufs[x], den_sp.at[dst_t.at[jnp.int32(it)]],
                             sd[x], add=True)
            if first_of_phase:
                @pl.when(i3 > 0)
                def _():
                    pltpu.make_async_copy(
                        rows[y], usum_sp.at[dst_t.at[jnp.int32(it - 1)]],
                        su[y]).wait()
                    pltpu.make_async_copy(
                        wbufs[y], den_sp.at[dst_t.at[jnp.int32(it - 1)]],
                        sd[y]).wait()
            else:
                pltpu.make_async_copy(
                    rows[y], usum_sp.at[dst_t.at[jnp.int32(it - 1)]],
                    su[y]).wait()
                pltpu.make_async_copy(
                    wbufs[y], den_sp.at[dst_t.at[jnp.int32(it - 1)]],
                    sd[y]).wait()

        gather(jnp.int32(0), 0)
        gather(jnp.int32(1), 1)

        def triple(i3, _):
            it0 = i3 * 3
            step(it0, 0, i3, True)
            gather(it0 + 2, 2)

            step(it0 + 1, 1, i3, False)

            @pl.when(i3 < (N_IT // 3) - 1)
            def _():
                gather(it0 + 3, 0)

            step(it0 + 2, 2, i3, False)

            @pl.when(i3 < (N_IT // 3) - 1)
            def _():
                gather(it0 + 4, 1)
            return 0

        lax.fori_loop(jnp.int32(0), jnp.int32(N_IT // 3), triple, 0)
        lx = (N_IT - 1) % 3
        pltpu.make_async_copy(rows[lx],
                              usum_sp.at[dst_t.at[jnp.int32(N_IT - 1)]],
                              su[lx]).wait()
        pltpu.make_async_copy(wbufs[lx],
                              den_sp.at[dst_t.at[jnp.int32(N_IT - 1)]],
                              sd[lx]).wait()
        plsc.subcore_barrier()

        for b in range(STRIPE // K):
            off = base + b * K
            pltpu.sync_copy(usum_sp.at[pl.ds(off, K)],
                            usum_out.at[c, pl.ds(off, K)])
            pltpu.sync_copy(den_sp.at[pl.ds(off, K)],
                            den_out.at[pl.ds(c * NP + off, K)])
        plsc.subcore_barrier()


@functools.cache
def _edge_kernel_fn():
  return pl.kernel(
    _edge_body,
    out_type=[jax.ShapeDtypeStruct((NC, NP, H), _f32),
              jax.ShapeDtypeStruct((NC, NP, H), _f32),
              jax.ShapeDtypeStruct((NC * NP,), _f32),
              jax.ShapeDtypeStruct((NC * NP,), _f32)],
    mesh=plsc.VectorSubcoreMesh(core_axis_name="c", subcore_axis_name="s"),
    compiler_params=pltpu.CompilerParams(needs_layout_passes=False,
                                         use_tc_tiling_on_sc=False),
    scratch_types=[
        pltpu.VMEM((N_IT, K), jnp.int32),
        pltpu.VMEM((N_IT, K), jnp.int32),
        pltpu.VMEM((NP,), _f32),
        pltpu.VMEM((NP,), _f32),
        pltpu.VMEM((K, H), _f32),
        pltpu.VMEM((K, H), _f32),
        pltpu.VMEM((K, H), _f32),
        pltpu.VMEM((K,), _f32),
        pltpu.VMEM((K,), _f32),
        pltpu.VMEM((K,), _f32),
        pltpu.VMEM((K, H), _f32),
        pltpu.VMEM((K,), _f32),
        pltpu.VMEM_SHARED((NP, H), _f32),
        pltpu.VMEM_SHARED((NP,), _f32),
    ] + [pltpu.SemaphoreType.DMA] * 9,
  )


def _dense_a_body(nf_ref, ea_ref, Wn_ref, avn_ref, We_ref, ave_ref,
                  hn_ref, sn_ref, he_ref, se_ref):
    hn = jnp.dot(nf_ref[...], Wn_ref[...],
                 preferred_element_type=_f32, precision=_HIGH)
    hn_ref[...] = hn
    sn_ref[...] = jnp.dot(hn, avn_ref[...],
                          preferred_element_type=_f32, precision=_HIGH)
    he = jnp.dot(ea_ref[...], We_ref[...],
                 preferred_element_type=_f32, precision=_HIGH)
    he_ref[...] = he
    se_ref[...] = jnp.dot(he, ave_ref[...],
                          preferred_element_type=_f32, precision=_HIGH)


_dense_a = pl.pallas_call(
    _dense_a_body,
    out_shape=[jax.ShapeDtypeStruct((NP, H), _f32),
               jax.ShapeDtypeStruct((NP, 2), _f32),
               jax.ShapeDtypeStruct((NP, H), _f32),
               jax.ShapeDtypeStruct((NP, 2), _f32)],
)


def _combine_bn(usum, den2, bias, gg, bb, relu):
    dd = jnp.transpose(jnp.maximum(den2[0:1, :] + den2[1:2, :], 1e-30))
    o = (usum[0] + usum[1]) / dd + bias
    if relu:
        o = jnp.maximum(o, 0.0)
    rows = lax.broadcasted_iota(jnp.int32, (NP, 1), 0)
    mask = rows < N
    o = jnp.where(mask, o, 0.0)
    mean = jnp.sum(o, axis=0, keepdims=True) / N
    var = jnp.sum(o * o, axis=0, keepdims=True) / N - mean * mean
    xn = (o - mean) / jnp.sqrt(var + 1e-5) * gg + bb
    return jnp.where(mask, xn, 0.0)


def _dense_b_body(usum_ref, den_ref, bias_ref, g_ref, bb_ref, W2_ref,
                  av2_ref, h2_ref, s2_ref):
    xn = _combine_bn(usum_ref[...], den_ref[...], bias_ref[...],
                     g_ref[...], bb_ref[...], relu=True)
    h2 = jnp.dot(xn, W2_ref[...], preferred_element_type=_f32,
                 precision=_HIGH)
    h2_ref[...] = h2
    s2_ref[...] = jnp.dot(h2, av2_ref[...], preferred_element_type=_f32,
                          precision=_HIGH)


_dense_b = pl.pallas_call(
    _dense_b_body,
    out_shape=[jax.ShapeDtypeStruct((NP, H), _f32),
               jax.ShapeDtypeStruct((NP, 2), _f32)],
)


def _pool(x, batch_ref):
    b = batch_ref[...]
    gids = lax.broadcasted_iota(jnp.int32, (1, GRP), 1)
    onehot = (b == gids).astype(_f32)
    sums = lax.dot_general(onehot, x, (((0,), (0,)), ((), ())),
                           preferred_element_type=_f32,
                           precision=_HIGH)
    cnt = jnp.transpose(jnp.sum(onehot, axis=0, keepdims=True))
    return sums / jnp.maximum(cnt, 1.0)


def _dense_cpart_body(usum_ref, den_ref, bias_ref, g_ref, bb_ref, batch_ref,
                      pool_ref):
    xh = _combine_bn(usum_ref[...], den_ref[...], bias_ref[...],
                     g_ref[...], bb_ref[...], relu=False)
    pool_ref[...] = _pool(xh, batch_ref)


_dense_cpart = pl.pallas_call(
    _dense_cpart_body,
    out_shape=jax.ShapeDtypeStruct((GRP, H), _f32),
)


def _dense_cfin_body(pn_ref, pe_ref, w_ref, Wm1_ref, bm1_ref, gw_ref, bw_ref,
                     Wm2_ref, bm2_ref, Wr1_ref, br1_ref, Wr2_ref, br2_ref,
                     out_ref):
    wh = jnp.dot(w_ref[...], Wm1_ref[...], preferred_element_type=_f32,
                 precision=_HIGH) + bm1_ref[...]
    wh = jnp.maximum(wh, 0.0)
    m = jnp.mean(wh, axis=0, keepdims=True)
    v = jnp.mean(wh * wh, axis=0, keepdims=True) - m * m
    wh = (wh - m) / jnp.sqrt(v + 1e-5) * gw_ref[...] + bw_ref[...]
    wh = jnp.dot(wh, Wm2_ref[...], preferred_element_type=_f32,
                 precision=_HIGH) + bm2_ref[...]
    comb = jnp.concatenate([pn_ref[...], pe_ref[...], wh], axis=-1)
    oh = jnp.dot(comb, Wr1_ref[...], preferred_element_type=_f32,
                 precision=_HIGH) + br1_ref[...]
    oh = jnp.maximum(oh, 0.0)
    out = jnp.dot(oh, Wr2_ref[...], preferred_element_type=_f32,
                  precision=_HIGH) + br2_ref[...]
    out_ref[...] = jax.nn.sigmoid(out)


_dense_cfin = pl.pallas_call(
    _dense_cfin_body,
    out_shape=jax.ShapeDtypeStruct((GRP, 1), _f32),
)


def kernel(node_features, edge_attr, weather_attr, edge_index, node_batch,
           edge_batch, Wn1, asn1, adn1, bn1, Wn2, asn2, adn2, bn2,
           We1, ase1, ade1, be1, We2, ase2, ade2, be2,
           Wm1, bm1, Wm2, bm2,
           g_nbn1, b_nbn1, g_nbn2, b_nbn2, g_ebn1, b_ebn1, g_ebn2, b_ebn2,
           g_wbn, b_wbn, Wr1, br1, Wr2, br2):
    f32 = _f32
    loops = jnp.arange(N, dtype=jnp.int32)
    src = jnp.concatenate([edge_index[0].astype(jnp.int32), loops])
    dst = jnp.concatenate([edge_index[1].astype(jnp.int32), loops])
    pad = T_ALL - src.shape[0]
    src = jnp.concatenate([src, jnp.zeros((pad,), jnp.int32)])
    dst = jnp.concatenate([dst, jnp.full((pad,), TRASH, jnp.int32)])
    src3 = src.reshape(NW, N_IT, K)
    dst3 = dst.reshape(NW, N_IT, K)

    nf_p = jnp.zeros((NP, node_features.shape[1]), f32).at[:N].set(
        node_features.astype(f32))
    ea_p = jnp.zeros((NP, edge_attr.shape[1]), f32).at[:N].set(
        edge_attr.astype(f32))
    nb_p = jnp.full((NP, 1), GRP, jnp.int32).at[:N, 0].set(
        node_batch.astype(jnp.int32))
    eb_p = jnp.full((NP, 1), GRP, jnp.int32).at[:N, 0].set(
        edge_batch.astype(jnp.int32))

    def row(v):
        return v.astype(f32).reshape(1, -1)

    def av(a, d):
        return jnp.stack([a.astype(f32), d.astype(f32)], axis=1)

    hn1, sn1, he1, se1 = _dense_a(nf_p, ea_p, Wn1.astype(f32), av(asn1, adn1),
                                  We1.astype(f32), av(ase1, ade1))
    un1, ue1, dn1, de1 = _edge_kernel_fn()(src3, dst3, hn1, he1,
                                      sn1[:, 0], sn1[:, 1],
                                      se1[:, 0], se1[:, 1])
    hn2, sn2 = _dense_b(un1, dn1.reshape(NC, NP),
                        row(bn1), row(g_nbn1), row(b_nbn1),
                        Wn2.astype(f32), av(asn2, adn2))
    he2, se2 = _dense_b(ue1, de1.reshape(NC, NP),
                        row(be1), row(g_ebn1), row(b_ebn1),
                        We2.astype(f32), av(ase2, ade2))
    un2, ue2, dn2, de2 = _edge_kernel_fn()(src3, dst3, hn2, he2,
                                      sn2[:, 0], sn2[:, 1],
                                      se2[:, 0], se2[:, 1])
    pn = _dense_cpart(un2, dn2.reshape(NC, NP),
                      row(bn2), row(g_nbn2), row(b_nbn2), nb_p)
    pe = _dense_cpart(ue2, de2.reshape(NC, NP),
                      row(be2), row(g_ebn2), row(b_ebn2), eb_p)
    out = _dense_cfin(
        pn, pe, weather_attr.astype(f32), Wm1.astype(f32), row(bm1),
        row(g_wbn), row(b_wbn), Wm2.astype(f32), row(bm2),
        Wr1.astype(f32), row(br1), Wr2.astype(f32), row(br2))
    return out
